# baseline (speedup 1.0000x reference)
import functools

import jax
import jax.numpy as jnp
from jax import lax
from jax.experimental import pallas as pl
from jax.experimental.pallas import tpu as pltpu
from jax.experimental.pallas import tpu_sc as plsc

N = 10000
E = 160000
D_IN = 256
H = 8
D_OUT = 64
D_EMB = H * D_OUT
SEM_HID = 128

_NC = 2
_NS = 16
_NW = _NC * _NS
_C = 128
_EPW = 5120
_ZSTRIDE = 624
_ZSPAN = 640
_RB = 1000


def _vbcast(v, lane):
    idx = jnp.full((16, 1), lane, dtype=jnp.int32)
    dnums = lax.GatherDimensionNumbers(
        offset_dims=(), collapsed_slice_dims=(0,), start_index_map=(0,))
    return lax.gather(v, idx, dnums, (1,),
                      mode=lax.GatherScatterMode.PROMISE_IN_BOUNDS)


def _vshift8(v):
    ii = lax.iota(jnp.int32, 16)
    idx = jnp.where(ii < 8, ii + 8, ii)[:, None]
    dnums = lax.GatherDimensionNumbers(
        offset_dims=(), collapsed_slice_dims=(0,), start_index_map=(0,))
    return lax.gather(v, idx, dnums, (1,),
                      mode=lax.GatherScatterMode.PROMISE_IN_BOUNDS)


def _stage1(x, W, Bmat):
    def body(x_ref, w_ref, b_ref, f4_ref, elr_ref):
        feat = jnp.dot(x_ref[...], w_ref[...],
                       preferred_element_type=jnp.float32)
        elr_ref[...] = jnp.dot(feat, b_ref[...],
                               preferred_element_type=jnp.float32)
        for j in range(4):
            f4_ref[j] = feat[:, j * 128:(j + 1) * 128]

    return pl.pallas_call(
        body,
        grid=(N // _RB,),
        in_specs=[
            pl.BlockSpec((_RB, D_IN), lambda i: (i, 0)),
            pl.BlockSpec((D_IN, D_EMB), lambda i: (0, 0)),
            pl.BlockSpec((D_EMB, 16), lambda i: (0, 0)),
        ],
        out_specs=[
            pl.BlockSpec((4, _RB, 128), lambda i: (0, i, 0)),
            pl.BlockSpec((_RB, 16), lambda i: (i, 0)),
        ],
        out_shape=[
            jax.ShapeDtypeStruct((4, N, 128), jnp.float32),
            jax.ShapeDtypeStruct((N, 16), jnp.float32),
        ],
    )(x, W, Bmat)


_CR = _EPW // _C
_NCROWS = (_NW * _EPW) // _C


def _sc_attention(lr, src2d, dst2d):
    mesh = plsc.VectorSubcoreMesh(core_axis_name="c", subcore_axis_name="s")

    @functools.partial(
        pl.kernel,
        out_type=[
            jax.ShapeDtypeStruct((E, 16), jnp.float32),
            jax.ShapeDtypeStruct((2 * N, 16), jnp.float32),
        ],
        mesh=mesh,
        compiler_params=pltpu.CompilerParams(use_tc_tiling_on_sc=False),
        scratch_types=[
            pltpu.VMEM((_CR, 128), jnp.int32),
            pltpu.VMEM((_CR, 128), jnp.int32),
            pltpu.VMEM((_C, 16), jnp.float32),
            pltpu.VMEM((_C, 16), jnp.float32),
            pltpu.VMEM((_C, 16), jnp.float32),
            pltpu.VMEM((_C, 16), jnp.float32),
            pltpu.VMEM((_ZSPAN, 16), jnp.float32),
            pltpu.VMEM_SHARED((N, 16), jnp.float32),
            pltpu.SemaphoreType.DMA,
            pltpu.SemaphoreType.DMA,
            pltpu.SemaphoreType.DMA,
        ],
    )
    def k(lr_hbm, src_hbm, dst_hbm, ex_hbm, den_hbm,
          src2, dst2, ga1, ga2, gb1, gb2, zb, acc, sem_a, sem_b, sem_s):
        c = lax.axis_index("c")
        s = lax.axis_index("s")
        rbase = (c * _NS + s) * _CR
        npr = jnp.minimum(E // _C - rbase, _CR) // 2

        pltpu.sync_copy(src_hbm.at[pl.ds(rbase, _CR)], src2)
        pltpu.sync_copy(dst_hbm.at[pl.ds(rbase, _CR)], dst2)

        def zrow(i, _):
            zb[i, :] = jnp.zeros((16,), jnp.float32)
            return 0
        lax.fori_loop(0, _ZSPAN, zrow, 0)
        pltpu.sync_copy(zb, acc.at[pl.ds(s * _ZSTRIDE, _ZSPAN)])
        plsc.subcore_barrier()

        def compute(gx, gy):
            def edge(i, _):
                v = gx[i, :] + _vshift8(gy[i, :])
                v = jnp.where(v > 0.0, v, 0.2 * v)
                gx[i, :] = jnp.exp(v)
                return 0
            lax.fori_loop(0, _C, edge, 0, unroll=4)

        def pair(p, _):
            ka, kb = 2 * p, 2 * p + 1
            ha1 = pltpu.async_copy(lr_hbm.at[src2.at[ka]], ga1, sem_a)
            ha2 = pltpu.async_copy(lr_hbm.at[dst2.at[ka]], ga2, sem_a)
            hb1 = pltpu.async_copy(lr_hbm.at[src2.at[kb]], gb1, sem_b)
            hb2 = pltpu.async_copy(lr_hbm.at[dst2.at[kb]], gb2, sem_b)
            ha1.wait(); ha2.wait()
            compute(ga1, ga2)
            hsa1 = pltpu.async_copy(
                ga1, ex_hbm.at[pl.ds((rbase + ka) * _C, _C)], sem_s)
            pltpu.sync_copy(ga1, acc.at[dst2.at[ka]], add=True)
            hb1.wait(); hb2.wait()
            compute(gb1, gb2)
            hsb1 = pltpu.async_copy(
                gb1, ex_hbm.at[pl.ds((rbase + kb) * _C, _C)], sem_s)
            pltpu.sync_copy(gb1, acc.at[dst2.at[kb]], add=True)
            hsa1.wait(); hsb1.wait()
            return 0
        lax.fori_loop(0, npr, pair, 0)

        plsc.subcore_barrier()
        pltpu.sync_copy(acc.at[pl.ds(s * _ZSTRIDE, _ZSPAN)], zb)
        pltpu.sync_copy(zb, den_hbm.at[pl.ds(c * N + s * _ZSTRIDE, _ZSPAN)])

    return k(lr, src2d, dst2d)


def _sc_aggregate(f0, f1, f2, f3, ex, src2d, dst2d):
    mesh = plsc.VectorSubcoreMesh(core_axis_name="c", subcore_axis_name="s")

    @functools.partial(
        pl.kernel,
        out_type=jax.ShapeDtypeStruct((4 * 2 * N, 128), jnp.float32),
        mesh=mesh,
        compiler_params=pltpu.CompilerParams(use_tc_tiling_on_sc=False),
        scratch_types=[
            pltpu.VMEM((_CR, 128), jnp.int32),
            pltpu.VMEM((_CR, 128), jnp.int32),
            pltpu.VMEM((_C, 128), jnp.float32),
            pltpu.VMEM((_C, 128), jnp.float32),
            pltpu.VMEM((_C, 16), jnp.float32),
            pltpu.VMEM((_C, 16), jnp.float32),
            pltpu.VMEM_SHARED((N, 128), jnp.float32),
            pltpu.SemaphoreType.DMA,
            pltpu.SemaphoreType.DMA,
            pltpu.SemaphoreType.DMA,
        ],
    )
    def k(f0_hbm, f1_hbm, f2_hbm, f3_hbm, ex_hbm, src_hbm, dst_hbm, u_hbm,
          src2, dst2, r0, r1, e0v, e1v, acc, sem_a, sem_b, sem_s):
        c = lax.axis_index("c")
        s = lax.axis_index("s")
        rbase = (c * _NS + s) * _CR
        npr = jnp.minimum(E // _C - rbase, _CR) // 2

        pltpu.sync_copy(src_hbm.at[pl.ds(rbase, _CR)], src2)
        pltpu.sync_copy(dst_hbm.at[pl.ds(rbase, _CR)], dst2)

        def compute(rows, exv, h0, h1):
            def edge(i, _):
                exr = exv[i, :]
                b0 = _vbcast(exr, h0)
                b1 = _vbcast(exr, h1)
                for j in range(4):
                    sl = pl.ds(j * 16, 16)
                    rows[i, sl] = rows[i, sl] * b0
                for j in range(4, 8):
                    sl = pl.ds(j * 16, 16)
                    rows[i, sl] = rows[i, sl] * b1
                return 0
            lax.fori_loop(0, _C, edge, 0, unroll=4)

        feats = [f0_hbm, f1_hbm, f2_hbm, f3_hbm]
        for fs in range(4):
            h0, h1 = 2 * fs, 2 * fs + 1

            def zrow(i, _):
                for j in range(8):
                    r0[i, pl.ds(j * 16, 16)] = jnp.zeros((16,), jnp.float32)
                return 0
            lax.fori_loop(0, 128, zrow, 0)
            for j in range(5):
                pltpu.sync_copy(r0, acc.at[pl.ds(s * _ZSTRIDE + j * 128, 128)])
            plsc.subcore_barrier()

            def pair(p, _):
                ka, kb = 2 * p, 2 * p + 1
                hga = pltpu.async_copy(feats[fs].at[src2.at[ka]], r0, sem_a)
                hea = pltpu.async_copy(
                    ex_hbm.at[pl.ds((rbase + ka) * _C, _C)], e0v, sem_a)
                hgb = pltpu.async_copy(feats[fs].at[src2.at[kb]], r1, sem_b)
                heb = pltpu.async_copy(
                    ex_hbm.at[pl.ds((rbase + kb) * _C, _C)], e1v, sem_b)
                hga.wait(); hea.wait()
                compute(r0, e0v, h0, h1)
                pltpu.sync_copy(r0, acc.at[dst2.at[ka]], add=True)
                hgb.wait(); heb.wait()
                compute(r1, e1v, h0, h1)
                pltpu.sync_copy(r1, acc.at[dst2.at[kb]], add=True)
                return 0
            lax.fori_loop(0, npr, pair, 0)

            plsc.subcore_barrier()
            ob = fs * 2 * N + c * N + s * _ZSTRIDE
            for j in range(5):
                pltpu.sync_copy(acc.at[pl.ds(s * _ZSTRIDE + j * 128, 128)], r0)
                pltpu.sync_copy(r0, u_hbm.at[pl.ds(ob + j * 128, 128)])
            plsc.subcore_barrier()

    return k(f0, f1, f2, f3, ex, src2d, dst2d)


def _stage4a(u, den, bvec, W1, b1r, W2p):
    def body(u_ref, d_ref, b_ref, w1_ref, b1_ref, w2_ref, z_ref, w_ref):
        recip = 1.0 / (d_ref[0] + d_ref[1] + 1e-9)
        zs = []
        for fs in range(4):
            Uj = u_ref[fs, 0] + u_ref[fs, 1]
            zs.append(Uj[:, :64] * recip[:, 2 * fs:2 * fs + 1])
            zs.append(Uj[:, 64:] * recip[:, 2 * fs + 1:2 * fs + 2])
        z = jnp.concatenate(zs, axis=1) + b_ref[...]
        z = jnp.where(z > 0.0, z, jnp.exp(z) - 1.0)
        z_ref[...] = z
        t = jnp.tanh(jnp.dot(z, w1_ref[...],
                             preferred_element_type=jnp.float32) + b1_ref[...])
        sc = jnp.sum(jnp.dot(t, w2_ref[...],
                             preferred_element_type=jnp.float32)) * (1.0 / N)

        @pl.when(pl.program_id(0) == 0)
        def _():
            w_ref[...] = jnp.zeros_like(w_ref)
        w_ref[...] += sc

    return pl.pallas_call(
        body,
        grid=(N // _RB,),
        in_specs=[
            pl.BlockSpec((4, 2, _RB, 128), lambda i: (0, 0, i, 0)),
            pl.BlockSpec((2, _RB, 16), lambda i: (0, i, 0)),
            pl.BlockSpec((1, D_EMB), lambda i: (0, 0)),
            pl.BlockSpec((D_EMB, SEM_HID), lambda i: (0, 0)),
            pl.BlockSpec((1, SEM_HID), lambda i: (0, 0)),
            pl.BlockSpec((SEM_HID, 8), lambda i: (0, 0)),
        ],
        out_specs=[
            pl.BlockSpec((_RB, D_EMB), lambda i: (i, 0)),
            pl.BlockSpec((8, 128), lambda i: (0, 0)),
        ],
        out_shape=[
            jax.ShapeDtypeStruct((N, D_EMB), jnp.float32),
            jax.ShapeDtypeStruct((8, 128), jnp.float32),
        ],
    )(u, den, bvec, W1, b1r, W2p)


def _stage4b(wpad, z0, z1, z2):
    def body(w_ref, z0_ref, z1_ref, z2_ref, o_ref):
        e = jnp.exp(w_ref[...])
        beta = e / jnp.sum(e, axis=0, keepdims=True)
        o_ref[...] = (z0_ref[...] * beta[0:1, 0:1]
                      + z1_ref[...] * beta[1:2, 0:1]
                      + z2_ref[...] * beta[2:3, 0:1])

    return pl.pallas_call(
        body,
        grid=(N // _RB,),
        in_specs=[
            pl.BlockSpec((8, 128), lambda i: (0, 0)),
            pl.BlockSpec((_RB, D_EMB), lambda i: (i, 0)),
            pl.BlockSpec((_RB, D_EMB), lambda i: (i, 0)),
            pl.BlockSpec((_RB, D_EMB), lambda i: (i, 0)),
        ],
        out_specs=pl.BlockSpec((_RB, D_EMB), lambda i: (i, 0)),
        out_shape=jax.ShapeDtypeStruct((N, D_EMB), jnp.float32),
    )(wpad, z0, z1, z2)


def _head_mats(al, ar):
    rows = jnp.arange(D_EMB)
    head = rows // D_OUT
    cols = jnp.arange(H)
    al_mat = jnp.where(cols[None, :] == head[:, None],
                       al.reshape(D_EMB)[:, None], 0.0)
    ar_mat = jnp.where(cols[None, :] == head[:, None],
                       ar.reshape(D_EMB)[:, None], 0.0)
    return jnp.concatenate([al_mat, ar_mat], axis=1).astype(jnp.float32)


def kernel(x0, x1, x2, edge_index0, edge_index1, edge_index2,
           W0, al0, ar0, b0, W1, al1, ar1, b1, W2, al2, ar2, b2,
           sem_W1, sem_b1, sem_W2):
    xs = [x0, x1, x2]
    eis = [edge_index0, edge_index1, edge_index2]
    Ws = [W0, W1, W2]
    als = [al0, al1, al2]
    ars = [ar0, ar1, ar2]
    bs = [b0, b1, b2]

    W2p = jnp.pad(sem_W2, ((0, 0), (0, 7)))
    b1r = sem_b1.reshape(1, SEM_HID)

    zlist, wlist = [], []
    npad = _NW * _EPW - E
    for i in range(3):
        src = jnp.pad(eis[i][0].astype(jnp.int32), (0, npad)).reshape(-1, _C)
        dst = jnp.pad(eis[i][1].astype(jnp.int32), (0, npad)).reshape(-1, _C)
        feat4, lr = _stage1(xs[i], Ws[i], _head_mats(als[i], ars[i]))
        ex, den = _sc_attention(lr, src, dst)
        u = _sc_aggregate(feat4[0], feat4[1], feat4[2], feat4[3],
                          ex, src, dst)
        z, w = _stage4a(u.reshape(4, 2, N, 128), den.reshape(2, N, 16),
                        bs[i].reshape(1, D_EMB), sem_W1, b1r, W2p)
        zlist.append(z)
        wlist.append(w)

    wpad = jnp.concatenate(
        [wlist[0][0:1, :], wlist[1][0:1, :], wlist[2][0:1, :],
         jnp.full((5, 128), -1e30, jnp.float32)], axis=0)
    return _stage4b(wpad, zlist[0], zlist[1], zlist[2])


# speedup vs baseline: 1.0060x; 1.0060x over previous
import functools

import jax
import jax.numpy as jnp
from jax import lax
from jax.experimental import pallas as pl
from jax.experimental.pallas import tpu as pltpu
from jax.experimental.pallas import tpu_sc as plsc

N = 10000
E = 160000
D_IN = 256
H = 8
D_OUT = 64
D_EMB = H * D_OUT
SEM_HID = 128

_NC = 2
_NS = 16
_NW = _NC * _NS
_C = 128
_EPW = 5120
_ZSTRIDE = 624
_ZSPAN = 640
_RB = 1000


def _vbcast(v, lane):
    idx = jnp.full((16, 1), lane, dtype=jnp.int32)
    dnums = lax.GatherDimensionNumbers(
        offset_dims=(), collapsed_slice_dims=(0,), start_index_map=(0,))
    return lax.gather(v, idx, dnums, (1,),
                      mode=lax.GatherScatterMode.PROMISE_IN_BOUNDS)


def _vshift8(v):
    ii = lax.iota(jnp.int32, 16)
    idx = jnp.where(ii < 8, ii + 8, ii)[:, None]
    dnums = lax.GatherDimensionNumbers(
        offset_dims=(), collapsed_slice_dims=(0,), start_index_map=(0,))
    return lax.gather(v, idx, dnums, (1,),
                      mode=lax.GatherScatterMode.PROMISE_IN_BOUNDS)


def _stage1(x, W, Bmat):
    def body(x_ref, w_ref, b_ref, f4_ref, elr_ref):
        feat = jnp.dot(x_ref[...], w_ref[...],
                       preferred_element_type=jnp.float32)
        elr_ref[...] = jnp.dot(feat, b_ref[...],
                               preferred_element_type=jnp.float32)
        for j in range(4):
            f4_ref[j] = feat[:, j * 128:(j + 1) * 128]

    return pl.pallas_call(
        body,
        grid=(N // _RB,),
        in_specs=[
            pl.BlockSpec((_RB, D_IN), lambda i: (i, 0)),
            pl.BlockSpec((D_IN, D_EMB), lambda i: (0, 0)),
            pl.BlockSpec((D_EMB, 16), lambda i: (0, 0)),
        ],
        out_specs=[
            pl.BlockSpec((4, _RB, 128), lambda i: (0, i, 0)),
            pl.BlockSpec((_RB, 16), lambda i: (i, 0)),
        ],
        out_shape=[
            jax.ShapeDtypeStruct((4, N, 128), jnp.float32),
            jax.ShapeDtypeStruct((N, 16), jnp.float32),
        ],
    )(x, W, Bmat)


_CR = _EPW // _C
_NCROWS = (_NW * _EPW) // _C


def _sc_attention(lr, src2d, dst2d):
    mesh = plsc.VectorSubcoreMesh(core_axis_name="c", subcore_axis_name="s")

    @functools.partial(
        pl.kernel,
        out_type=[
            jax.ShapeDtypeStruct((E, 16), jnp.float32),
            jax.ShapeDtypeStruct((2 * N, 16), jnp.float32),
        ],
        mesh=mesh,
        compiler_params=pltpu.CompilerParams(use_tc_tiling_on_sc=False),
        scratch_types=[
            pltpu.VMEM((_CR, 128), jnp.int32),
            pltpu.VMEM((_CR, 128), jnp.int32),
            pltpu.VMEM((_C, 16), jnp.float32),
            pltpu.VMEM((_C, 16), jnp.float32),
            pltpu.VMEM((_C, 16), jnp.float32),
            pltpu.VMEM((_C, 16), jnp.float32),
            pltpu.VMEM((_ZSPAN, 16), jnp.float32),
            pltpu.VMEM_SHARED((N, 16), jnp.float32),
            pltpu.SemaphoreType.DMA,
            pltpu.SemaphoreType.DMA,
            pltpu.SemaphoreType.DMA,
        ],
    )
    def k(lr_hbm, src_hbm, dst_hbm, ex_hbm, den_hbm,
          src2, dst2, ga1, ga2, gb1, gb2, zb, acc, sem_a, sem_b, sem_s):
        c = lax.axis_index("c")
        s = lax.axis_index("s")
        rbase = (c * _NS + s) * _CR
        npr = jnp.minimum(E // _C - rbase, _CR) // 2

        pltpu.sync_copy(src_hbm.at[pl.ds(rbase, _CR)], src2)
        pltpu.sync_copy(dst_hbm.at[pl.ds(rbase, _CR)], dst2)

        def zrow(i, _):
            zb[i, :] = jnp.zeros((16,), jnp.float32)
            return 0
        lax.fori_loop(0, _ZSPAN, zrow, 0)
        pltpu.sync_copy(zb, acc.at[pl.ds(s * _ZSTRIDE, _ZSPAN)])
        plsc.subcore_barrier()

        def compute(gx, gy):
            def edge(i, _):
                v = gx[i, :] + _vshift8(gy[i, :])
                v = jnp.where(v > 0.0, v, 0.2 * v)
                gx[i, :] = jnp.exp(v)
                return 0
            lax.fori_loop(0, _C, edge, 0, unroll=2)

        def pair(p, _):
            ka, kb = 2 * p, 2 * p + 1
            ha1 = pltpu.async_copy(lr_hbm.at[src2.at[ka]], ga1, sem_a)
            ha2 = pltpu.async_copy(lr_hbm.at[dst2.at[ka]], ga2, sem_a)
            hb1 = pltpu.async_copy(lr_hbm.at[src2.at[kb]], gb1, sem_b)
            hb2 = pltpu.async_copy(lr_hbm.at[dst2.at[kb]], gb2, sem_b)
            ha1.wait(); ha2.wait()
            compute(ga1, ga2)
            hsa1 = pltpu.async_copy(
                ga1, ex_hbm.at[pl.ds((rbase + ka) * _C, _C)], sem_s)
            pltpu.sync_copy(ga1, acc.at[dst2.at[ka]], add=True)
            hb1.wait(); hb2.wait()
            compute(gb1, gb2)
            hsb1 = pltpu.async_copy(
                gb1, ex_hbm.at[pl.ds((rbase + kb) * _C, _C)], sem_s)
            pltpu.sync_copy(gb1, acc.at[dst2.at[kb]], add=True)
            hsa1.wait(); hsb1.wait()
            return 0
        lax.fori_loop(0, npr, pair, 0)

        plsc.subcore_barrier()
        pltpu.sync_copy(acc.at[pl.ds(s * _ZSTRIDE, _ZSPAN)], zb)
        pltpu.sync_copy(zb, den_hbm.at[pl.ds(c * N + s * _ZSTRIDE, _ZSPAN)])

    return k(lr, src2d, dst2d)


def _sc_aggregate(f0, f1, f2, f3, ex, src2d, dst2d):
    mesh = plsc.VectorSubcoreMesh(core_axis_name="c", subcore_axis_name="s")

    @functools.partial(
        pl.kernel,
        out_type=jax.ShapeDtypeStruct((4 * 2 * N, 128), jnp.float32),
        mesh=mesh,
        compiler_params=pltpu.CompilerParams(use_tc_tiling_on_sc=False),
        scratch_types=[
            pltpu.VMEM((_CR, 128), jnp.int32),
            pltpu.VMEM((_CR, 128), jnp.int32),
            pltpu.VMEM((_C, 128), jnp.float32),
            pltpu.VMEM((_C, 128), jnp.float32),
            pltpu.VMEM((_C, 16), jnp.float32),
            pltpu.VMEM((_C, 16), jnp.float32),
            pltpu.VMEM_SHARED((N, 128), jnp.float32),
            pltpu.SemaphoreType.DMA,
            pltpu.SemaphoreType.DMA,
            pltpu.SemaphoreType.DMA,
        ],
    )
    def k(f0_hbm, f1_hbm, f2_hbm, f3_hbm, ex_hbm, src_hbm, dst_hbm, u_hbm,
          src2, dst2, r0, r1, e0v, e1v, acc, sem_a, sem_b, sem_s):
        c = lax.axis_index("c")
        s = lax.axis_index("s")
        rbase = (c * _NS + s) * _CR
        npr = jnp.minimum(E // _C - rbase, _CR) // 2

        pltpu.sync_copy(src_hbm.at[pl.ds(rbase, _CR)], src2)
        pltpu.sync_copy(dst_hbm.at[pl.ds(rbase, _CR)], dst2)

        def compute(rows, exv, h0, h1):
            def edge(i, _):
                exr = exv[i, :]
                b0 = _vbcast(exr, h0)
                b1 = _vbcast(exr, h1)
                for j in range(4):
                    sl = pl.ds(j * 16, 16)
                    rows[i, sl] = rows[i, sl] * b0
                for j in range(4, 8):
                    sl = pl.ds(j * 16, 16)
                    rows[i, sl] = rows[i, sl] * b1
                return 0
            lax.fori_loop(0, _C, edge, 0, unroll=2)

        feats = [f0_hbm, f1_hbm, f2_hbm, f3_hbm]
        for fs in range(4):
            h0, h1 = 2 * fs, 2 * fs + 1

            def zrow(i, _):
                for j in range(8):
                    r0[i, pl.ds(j * 16, 16)] = jnp.zeros((16,), jnp.float32)
                return 0
            lax.fori_loop(0, 128, zrow, 0)
            for j in range(5):
                pltpu.sync_copy(r0, acc.at[pl.ds(s * _ZSTRIDE + j * 128, 128)])
            plsc.subcore_barrier()

            def pair(p, _):
                ka, kb = 2 * p, 2 * p + 1
                hga = pltpu.async_copy(feats[fs].at[src2.at[ka]], r0, sem_a)
                hea = pltpu.async_copy(
                    ex_hbm.at[pl.ds((rbase + ka) * _C, _C)], e0v, sem_a)
                hgb = pltpu.async_copy(feats[fs].at[src2.at[kb]], r1, sem_b)
                heb = pltpu.async_copy(
                    ex_hbm.at[pl.ds((rbase + kb) * _C, _C)], e1v, sem_b)
                hga.wait(); hea.wait()
                compute(r0, e0v, h0, h1)
                pltpu.sync_copy(r0, acc.at[dst2.at[ka]], add=True)
                hgb.wait(); heb.wait()
                compute(r1, e1v, h0, h1)
                pltpu.sync_copy(r1, acc.at[dst2.at[kb]], add=True)
                return 0
            lax.fori_loop(0, npr, pair, 0)

            plsc.subcore_barrier()
            ob = fs * 2 * N + c * N + s * _ZSTRIDE
            for j in range(5):
                pltpu.sync_copy(acc.at[pl.ds(s * _ZSTRIDE + j * 128, 128)], r0)
                pltpu.sync_copy(r0, u_hbm.at[pl.ds(ob + j * 128, 128)])
            plsc.subcore_barrier()

    return k(f0, f1, f2, f3, ex, src2d, dst2d)


def _stage4a(u, den, bvec, W1, b1r, W2p):
    def body(u_ref, d_ref, b_ref, w1_ref, b1_ref, w2_ref, z_ref, w_ref):
        recip = 1.0 / (d_ref[0] + d_ref[1] + 1e-9)
        zs = []
        for fs in range(4):
            Uj = u_ref[fs, 0] + u_ref[fs, 1]
            zs.append(Uj[:, :64] * recip[:, 2 * fs:2 * fs + 1])
            zs.append(Uj[:, 64:] * recip[:, 2 * fs + 1:2 * fs + 2])
        z = jnp.concatenate(zs, axis=1) + b_ref[...]
        z = jnp.where(z > 0.0, z, jnp.exp(z) - 1.0)
        z_ref[...] = z
        t = jnp.tanh(jnp.dot(z, w1_ref[...],
                             preferred_element_type=jnp.float32) + b1_ref[...])
        sc = jnp.sum(jnp.dot(t, w2_ref[...],
                             preferred_element_type=jnp.float32)) * (1.0 / N)

        @pl.when(pl.program_id(0) == 0)
        def _():
            w_ref[...] = jnp.zeros_like(w_ref)
        w_ref[...] += sc

    return pl.pallas_call(
        body,
        grid=(N // _RB,),
        in_specs=[
            pl.BlockSpec((4, 2, _RB, 128), lambda i: (0, 0, i, 0)),
            pl.BlockSpec((2, _RB, 16), lambda i: (0, i, 0)),
            pl.BlockSpec((1, D_EMB), lambda i: (0, 0)),
            pl.BlockSpec((D_EMB, SEM_HID), lambda i: (0, 0)),
            pl.BlockSpec((1, SEM_HID), lambda i: (0, 0)),
            pl.BlockSpec((SEM_HID, 8), lambda i: (0, 0)),
        ],
        out_specs=[
            pl.BlockSpec((_RB, D_EMB), lambda i: (i, 0)),
            pl.BlockSpec((8, 128), lambda i: (0, 0)),
        ],
        out_shape=[
            jax.ShapeDtypeStruct((N, D_EMB), jnp.float32),
            jax.ShapeDtypeStruct((8, 128), jnp.float32),
        ],
    )(u, den, bvec, W1, b1r, W2p)


def _stage4b(wpad, z0, z1, z2):
    def body(w_ref, z0_ref, z1_ref, z2_ref, o_ref):
        e = jnp.exp(w_ref[...])
        beta = e / jnp.sum(e, axis=0, keepdims=True)
        o_ref[...] = (z0_ref[...] * beta[0:1, 0:1]
                      + z1_ref[...] * beta[1:2, 0:1]
                      + z2_ref[...] * beta[2:3, 0:1])

    return pl.pallas_call(
        body,
        grid=(N // _RB,),
        in_specs=[
            pl.BlockSpec((8, 128), lambda i: (0, 0)),
            pl.BlockSpec((_RB, D_EMB), lambda i: (i, 0)),
            pl.BlockSpec((_RB, D_EMB), lambda i: (i, 0)),
            pl.BlockSpec((_RB, D_EMB), lambda i: (i, 0)),
        ],
        out_specs=pl.BlockSpec((_RB, D_EMB), lambda i: (i, 0)),
        out_shape=jax.ShapeDtypeStruct((N, D_EMB), jnp.float32),
    )(wpad, z0, z1, z2)


def _head_mats(al, ar):
    rows = jnp.arange(D_EMB)
    head = rows // D_OUT
    cols = jnp.arange(H)
    al_mat = jnp.where(cols[None, :] == head[:, None],
                       al.reshape(D_EMB)[:, None], 0.0)
    ar_mat = jnp.where(cols[None, :] == head[:, None],
                       ar.reshape(D_EMB)[:, None], 0.0)
    return jnp.concatenate([al_mat, ar_mat], axis=1).astype(jnp.float32)


def kernel(x0, x1, x2, edge_index0, edge_index1, edge_index2,
           W0, al0, ar0, b0, W1, al1, ar1, b1, W2, al2, ar2, b2,
           sem_W1, sem_b1, sem_W2):
    xs = [x0, x1, x2]
    eis = [edge_index0, edge_index1, edge_index2]
    Ws = [W0, W1, W2]
    als = [al0, al1, al2]
    ars = [ar0, ar1, ar2]
    bs = [b0, b1, b2]

    W2p = jnp.pad(sem_W2, ((0, 0), (0, 7)))
    b1r = sem_b1.reshape(1, SEM_HID)

    zlist, wlist = [], []
    npad = _NW * _EPW - E
    for i in range(3):
        src = jnp.pad(eis[i][0].astype(jnp.int32), (0, npad)).reshape(-1, _C)
        dst = jnp.pad(eis[i][1].astype(jnp.int32), (0, npad)).reshape(-1, _C)
        feat4, lr = _stage1(xs[i], Ws[i], _head_mats(als[i], ars[i]))
        ex, den = _sc_attention(lr, src, dst)
        u = _sc_aggregate(feat4[0], feat4[1], feat4[2], feat4[3],
                          ex, src, dst)
        z, w = _stage4a(u.reshape(4, 2, N, 128), den.reshape(2, N, 16),
                        bs[i].reshape(1, D_EMB), sem_W1, b1r, W2p)
        zlist.append(z)
        wlist.append(w)

    wpad = jnp.concatenate(
        [wlist[0][0:1, :], wlist[1][0:1, :], wlist[2][0:1, :],
         jnp.full((5, 128), -1e30, jnp.float32)], axis=0)
    return _stage4b(wpad, zlist[0], zlist[1], zlist[2])


# speedup vs baseline: 1.0877x; 1.0812x over previous
import functools

import jax
import jax.numpy as jnp
from jax import lax
from jax.experimental import pallas as pl
from jax.experimental.pallas import tpu as pltpu
from jax.experimental.pallas import tpu_sc as plsc

N = 10000
E = 160000
D_IN = 256
H = 8
D_OUT = 64
D_EMB = H * D_OUT
SEM_HID = 128

_NC = 2
_NS = 16
_NW = _NC * _NS
_C = 128
_EPW = 5120
_ZSTRIDE = 624
_ZSPAN = 640
_RB = 1000


def _vbcast(v, lane):
    idx = jnp.full((16, 1), lane, dtype=jnp.int32)
    dnums = lax.GatherDimensionNumbers(
        offset_dims=(), collapsed_slice_dims=(0,), start_index_map=(0,))
    return lax.gather(v, idx, dnums, (1,),
                      mode=lax.GatherScatterMode.PROMISE_IN_BOUNDS)


def _vshift8(v):
    ii = lax.iota(jnp.int32, 16)
    idx = jnp.where(ii < 8, ii + 8, ii)[:, None]
    dnums = lax.GatherDimensionNumbers(
        offset_dims=(), collapsed_slice_dims=(0,), start_index_map=(0,))
    return lax.gather(v, idx, dnums, (1,),
                      mode=lax.GatherScatterMode.PROMISE_IN_BOUNDS)


def _stage1(x, W, Bmat):
    def body(x_ref, w_ref, b_ref, f4_ref, elr_ref):
        feat = jnp.dot(x_ref[...], w_ref[...],
                       preferred_element_type=jnp.float32)
        elr_ref[...] = jnp.dot(feat, b_ref[...],
                               preferred_element_type=jnp.float32)
        for j in range(4):
            f4_ref[j] = feat[:, j * 128:(j + 1) * 128]

    return pl.pallas_call(
        body,
        grid=(N // _RB,),
        in_specs=[
            pl.BlockSpec((_RB, D_IN), lambda i: (i, 0)),
            pl.BlockSpec((D_IN, D_EMB), lambda i: (0, 0)),
            pl.BlockSpec((D_EMB, 16), lambda i: (0, 0)),
        ],
        out_specs=[
            pl.BlockSpec((4, _RB, 128), lambda i: (0, i, 0)),
            pl.BlockSpec((_RB, 16), lambda i: (i, 0)),
        ],
        out_shape=[
            jax.ShapeDtypeStruct((4, N, 128), jnp.float32),
            jax.ShapeDtypeStruct((N, 16), jnp.float32),
        ],
    )(x, W, Bmat)


_CR = _EPW // _C
_NCROWS = (_NW * _EPW) // _C


def _sc_attention(lr, src2d, dst2d):
    mesh = plsc.VectorSubcoreMesh(core_axis_name="c", subcore_axis_name="s")

    @functools.partial(
        pl.kernel,
        out_type=[
            jax.ShapeDtypeStruct((E, 16), jnp.float32),
            jax.ShapeDtypeStruct((2 * N, 16), jnp.float32),
        ],
        mesh=mesh,
        compiler_params=pltpu.CompilerParams(use_tc_tiling_on_sc=False),
        scratch_types=[
            pltpu.VMEM((_CR, 128), jnp.int32),
            pltpu.VMEM((_CR, 128), jnp.int32),
            pltpu.VMEM((_C, 16), jnp.float32),
            pltpu.VMEM((_C, 16), jnp.float32),
            pltpu.VMEM((_C, 16), jnp.float32),
            pltpu.VMEM((_C, 16), jnp.float32),
            pltpu.VMEM((_ZSPAN, 16), jnp.float32),
            pltpu.VMEM_SHARED((N, 16), jnp.float32),
            pltpu.SemaphoreType.DMA,
            pltpu.SemaphoreType.DMA,
            pltpu.SemaphoreType.DMA,
        ],
    )
    def k(lr_hbm, src_hbm, dst_hbm, ex_hbm, den_hbm,
          src2, dst2, ga1, ga2, gb1, gb2, zb, acc, sem_a, sem_b, sem_s):
        c = lax.axis_index("c")
        s = lax.axis_index("s")
        rbase = (c * _NS + s) * _CR
        npr = jnp.minimum(E // _C - rbase, _CR) // 2

        pltpu.sync_copy(src_hbm.at[pl.ds(rbase, _CR)], src2)
        pltpu.sync_copy(dst_hbm.at[pl.ds(rbase, _CR)], dst2)

        def zrow(i, _):
            zb[i, :] = jnp.zeros((16,), jnp.float32)
            return 0
        lax.fori_loop(0, _ZSPAN, zrow, 0)
        pltpu.sync_copy(zb, acc.at[pl.ds(s * _ZSTRIDE, _ZSPAN)])
        plsc.subcore_barrier()

        def compute(gx, gy):
            def edge(i, _):
                v = gx[i, :] + _vshift8(gy[i, :])
                v = jnp.where(v > 0.0, v, 0.2 * v)
                gx[i, :] = jnp.exp(v)
                return 0
            lax.fori_loop(0, _C, edge, 0, unroll=2)

        def pair(p, _):
            ka, kb = 2 * p, 2 * p + 1
            ha1 = pltpu.async_copy(lr_hbm.at[src2.at[ka]], ga1, sem_a)
            ha2 = pltpu.async_copy(lr_hbm.at[dst2.at[ka]], ga2, sem_a)
            hb1 = pltpu.async_copy(lr_hbm.at[src2.at[kb]], gb1, sem_b)
            hb2 = pltpu.async_copy(lr_hbm.at[dst2.at[kb]], gb2, sem_b)
            ha1.wait(); ha2.wait()
            compute(ga1, ga2)
            hsa1 = pltpu.async_copy(
                ga1, ex_hbm.at[pl.ds((rbase + ka) * _C, _C)], sem_s)
            pltpu.sync_copy(ga1, acc.at[dst2.at[ka]], add=True)
            hb1.wait(); hb2.wait()
            compute(gb1, gb2)
            hsb1 = pltpu.async_copy(
                gb1, ex_hbm.at[pl.ds((rbase + kb) * _C, _C)], sem_s)
            pltpu.sync_copy(gb1, acc.at[dst2.at[kb]], add=True)
            hsa1.wait(); hsb1.wait()
            return 0
        lax.fori_loop(0, npr, pair, 0)

        plsc.subcore_barrier()
        pltpu.sync_copy(acc.at[pl.ds(s * _ZSTRIDE, _ZSPAN)], zb)
        pltpu.sync_copy(zb, den_hbm.at[pl.ds(c * N + s * _ZSTRIDE, _ZSPAN)])

    return k(lr, src2d, dst2d)


def _sc_aggregate(f0, f1, f2, f3, ex, src2d, dst2d):
    mesh = plsc.VectorSubcoreMesh(core_axis_name="c", subcore_axis_name="s")

    @functools.partial(
        pl.kernel,
        out_type=jax.ShapeDtypeStruct((4 * 2 * N, 128), jnp.float32),
        mesh=mesh,
        compiler_params=pltpu.CompilerParams(use_tc_tiling_on_sc=False),
        scratch_types=[
            pltpu.VMEM((_CR, 128), jnp.int32),
            pltpu.VMEM((_CR, 128), jnp.int32),
            pltpu.VMEM((_C, 128), jnp.float32),
            pltpu.VMEM((_C, 128), jnp.float32),
            pltpu.VMEM((_C, 16), jnp.float32),
            pltpu.VMEM((_C, 16), jnp.float32),
            pltpu.VMEM_SHARED((N, 128), jnp.float32),
            pltpu.SemaphoreType.DMA,
            pltpu.SemaphoreType.DMA,
            pltpu.SemaphoreType.DMA,
        ],
    )
    def k(f0_hbm, f1_hbm, f2_hbm, f3_hbm, ex_hbm, src_hbm, dst_hbm, u_hbm,
          src2, dst2, r0, r1, e0v, e1v, acc, sem_a, sem_b, sem_s):
        c = lax.axis_index("c")
        s = lax.axis_index("s")
        rbase = (c * _NS + s) * _CR
        npr = jnp.minimum(E // _C - rbase, _CR) // 2

        pltpu.sync_copy(src_hbm.at[pl.ds(rbase, _CR)], src2)
        pltpu.sync_copy(dst_hbm.at[pl.ds(rbase, _CR)], dst2)

        def compute(rows, exv, h0, h1):
            def edge(i, _):
                exr = exv[i, :]
                b0 = _vbcast(exr, h0)
                b1 = _vbcast(exr, h1)
                for j in range(4):
                    sl = pl.ds(j * 16, 16)
                    rows[i, sl] = rows[i, sl] * b0
                for j in range(4, 8):
                    sl = pl.ds(j * 16, 16)
                    rows[i, sl] = rows[i, sl] * b1
                return 0
            lax.fori_loop(0, _C, edge, 0, unroll=2)

        feats = [f0_hbm, f1_hbm, f2_hbm, f3_hbm]
        for fs in range(4):
            h0, h1 = 2 * fs, 2 * fs + 1

            def zrow(i, _):
                for j in range(8):
                    r0[i, pl.ds(j * 16, 16)] = jnp.zeros((16,), jnp.float32)
                return 0
            lax.fori_loop(0, 128, zrow, 0)
            for j in range(5):
                pltpu.sync_copy(r0, acc.at[pl.ds(s * _ZSTRIDE + j * 128, 128)])
            plsc.subcore_barrier()

            def pair(p, _):
                ka, kb = 2 * p, 2 * p + 1
                hga = pltpu.async_copy(feats[fs].at[src2.at[ka]], r0, sem_a)
                hea = pltpu.async_copy(
                    ex_hbm.at[pl.ds((rbase + ka) * _C, _C)], e0v, sem_a)
                hgb = pltpu.async_copy(feats[fs].at[src2.at[kb]], r1, sem_b)
                heb = pltpu.async_copy(
                    ex_hbm.at[pl.ds((rbase + kb) * _C, _C)], e1v, sem_b)
                hga.wait(); hea.wait()
                compute(r0, e0v, h0, h1)
                hsa = pltpu.async_copy(r0, acc.at[dst2.at[ka]], sem_s,
                                       add=True)
                hgb.wait(); heb.wait()
                compute(r1, e1v, h0, h1)
                hsa.wait()
                pltpu.sync_copy(r1, acc.at[dst2.at[kb]], add=True)
                return 0
            lax.fori_loop(0, npr, pair, 0)

            plsc.subcore_barrier()
            ob = fs * 2 * N + c * N + s * _ZSTRIDE
            for j in range(5):
                pltpu.sync_copy(acc.at[pl.ds(s * _ZSTRIDE + j * 128, 128)], r0)
                pltpu.sync_copy(r0, u_hbm.at[pl.ds(ob + j * 128, 128)])
            plsc.subcore_barrier()

    return k(f0, f1, f2, f3, ex, src2d, dst2d)


def _stage4a(u, den, bvec, W1, b1r, W2p):
    def body(u_ref, d_ref, b_ref, w1_ref, b1_ref, w2_ref, z_ref, w_ref):
        recip = 1.0 / (d_ref[0] + d_ref[1] + 1e-9)
        zs = []
        for fs in range(4):
            Uj = u_ref[fs, 0] + u_ref[fs, 1]
            zs.append(Uj[:, :64] * recip[:, 2 * fs:2 * fs + 1])
            zs.append(Uj[:, 64:] * recip[:, 2 * fs + 1:2 * fs + 2])
        z = jnp.concatenate(zs, axis=1) + b_ref[...]
        z = jnp.where(z > 0.0, z, jnp.exp(z) - 1.0)
        z_ref[...] = z
        t = jnp.tanh(jnp.dot(z, w1_ref[...],
                             preferred_element_type=jnp.float32) + b1_ref[...])
        sc = jnp.sum(jnp.dot(t, w2_ref[...],
                             preferred_element_type=jnp.float32)) * (1.0 / N)

        @pl.when(pl.program_id(0) == 0)
        def _():
            w_ref[...] = jnp.zeros_like(w_ref)
        w_ref[...] += sc

    return pl.pallas_call(
        body,
        grid=(N // _RB,),
        in_specs=[
            pl.BlockSpec((4, 2, _RB, 128), lambda i: (0, 0, i, 0)),
            pl.BlockSpec((2, _RB, 16), lambda i: (0, i, 0)),
            pl.BlockSpec((1, D_EMB), lambda i: (0, 0)),
            pl.BlockSpec((D_EMB, SEM_HID), lambda i: (0, 0)),
            pl.BlockSpec((1, SEM_HID), lambda i: (0, 0)),
            pl.BlockSpec((SEM_HID, 8), lambda i: (0, 0)),
        ],
        out_specs=[
            pl.BlockSpec((_RB, D_EMB), lambda i: (i, 0)),
            pl.BlockSpec((8, 128), lambda i: (0, 0)),
        ],
        out_shape=[
            jax.ShapeDtypeStruct((N, D_EMB), jnp.float32),
            jax.ShapeDtypeStruct((8, 128), jnp.float32),
        ],
    )(u, den, bvec, W1, b1r, W2p)


def _stage4b(wpad, z0, z1, z2):
    def body(w_ref, z0_ref, z1_ref, z2_ref, o_ref):
        e = jnp.exp(w_ref[...])
        beta = e / jnp.sum(e, axis=0, keepdims=True)
        o_ref[...] = (z0_ref[...] * beta[0:1, 0:1]
                      + z1_ref[...] * beta[1:2, 0:1]
                      + z2_ref[...] * beta[2:3, 0:1])

    return pl.pallas_call(
        body,
        grid=(N // _RB,),
        in_specs=[
            pl.BlockSpec((8, 128), lambda i: (0, 0)),
            pl.BlockSpec((_RB, D_EMB), lambda i: (i, 0)),
            pl.BlockSpec((_RB, D_EMB), lambda i: (i, 0)),
            pl.BlockSpec((_RB, D_EMB), lambda i: (i, 0)),
        ],
        out_specs=pl.BlockSpec((_RB, D_EMB), lambda i: (i, 0)),
        out_shape=jax.ShapeDtypeStruct((N, D_EMB), jnp.float32),
    )(wpad, z0, z1, z2)


def _head_mats(al, ar):
    rows = jnp.arange(D_EMB)
    head = rows // D_OUT
    cols = jnp.arange(H)
    al_mat = jnp.where(cols[None, :] == head[:, None],
                       al.reshape(D_EMB)[:, None], 0.0)
    ar_mat = jnp.where(cols[None, :] == head[:, None],
                       ar.reshape(D_EMB)[:, None], 0.0)
    return jnp.concatenate([al_mat, ar_mat], axis=1).astype(jnp.float32)


def kernel(x0, x1, x2, edge_index0, edge_index1, edge_index2,
           W0, al0, ar0, b0, W1, al1, ar1, b1, W2, al2, ar2, b2,
           sem_W1, sem_b1, sem_W2):
    xs = [x0, x1, x2]
    eis = [edge_index0, edge_index1, edge_index2]
    Ws = [W0, W1, W2]
    als = [al0, al1, al2]
    ars = [ar0, ar1, ar2]
    bs = [b0, b1, b2]

    W2p = jnp.pad(sem_W2, ((0, 0), (0, 7)))
    b1r = sem_b1.reshape(1, SEM_HID)

    zlist, wlist = [], []
    npad = _NW * _EPW - E
    for i in range(3):
        src = jnp.pad(eis[i][0].astype(jnp.int32), (0, npad)).reshape(-1, _C)
        dst = jnp.pad(eis[i][1].astype(jnp.int32), (0, npad)).reshape(-1, _C)
        feat4, lr = _stage1(xs[i], Ws[i], _head_mats(als[i], ars[i]))
        ex, den = _sc_attention(lr, src, dst)
        u = _sc_aggregate(feat4[0], feat4[1], feat4[2], feat4[3],
                          ex, src, dst)
        z, w = _stage4a(u.reshape(4, 2, N, 128), den.reshape(2, N, 16),
                        bs[i].reshape(1, D_EMB), sem_W1, b1r, W2p)
        zlist.append(z)
        wlist.append(w)

    wpad = jnp.concatenate(
        [wlist[0][0:1, :], wlist[1][0:1, :], wlist[2][0:1, :],
         jnp.full((5, 128), -1e30, jnp.float32)], axis=0)
    return _stage4b(wpad, zlist[0], zlist[1], zlist[2])
